# baseline (speedup 1.0000x reference)
import functools

import jax
import jax.numpy as jnp
from jax import lax
from jax.experimental import pallas as pl
from jax.experimental.pallas import tpu as pltpu
from jax.experimental.pallas import tpu_sc as plsc


def kernel(x, embedding):
    seq_len = x.shape[1]
    d_model = embedding.shape[1]
    info = plsc.get_sparse_core_info()
    num_workers = info.num_cores * info.num_subcores
    rows_per_worker = seq_len // num_workers

    mesh = plsc.VectorSubcoreMesh(core_axis_name="c", subcore_axis_name="s")

    @functools.partial(
        pl.kernel,
        mesh=mesh,
        out_type=jax.ShapeDtypeStruct((seq_len, d_model), embedding.dtype),
    )
    def _sc_copy(emb_hbm, out_hbm):
        wid = lax.axis_index("s") * info.num_cores + lax.axis_index("c")
        base = wid * rows_per_worker
        pltpu.sync_copy(
            emb_hbm.at[pl.ds(base, rows_per_worker)],
            out_hbm.at[pl.ds(base, rows_per_worker)],
        )

    return _sc_copy(embedding)[None]


# speedup vs baseline: 21.3047x; 21.3047x over previous
import functools

import jax
import jax.numpy as jnp
from jax import lax
from jax.experimental import pallas as pl
from jax.experimental.pallas import tpu as pltpu
from jax.experimental.pallas import tpu_sc as plsc

_CHUNK_ROWS = 64


def kernel(x, embedding):
    seq_len = x.shape[1]
    d_model = embedding.shape[1]
    info = plsc.get_sparse_core_info()
    num_workers = info.num_cores * info.num_subcores
    rows_per_worker = seq_len // num_workers
    n_chunks = rows_per_worker // _CHUNK_ROWS

    mesh = plsc.VectorSubcoreMesh(core_axis_name="c", subcore_axis_name="s")

    @functools.partial(
        pl.kernel,
        mesh=mesh,
        out_type=jax.ShapeDtypeStruct((seq_len, d_model), embedding.dtype),
        scratch_types=[
            pltpu.VMEM((2, _CHUNK_ROWS, d_model), embedding.dtype),
            pltpu.SemaphoreType.DMA,
            pltpu.SemaphoreType.DMA,
        ],
    )
    def _sc_copy(emb_hbm, out_hbm, buf, sem_in, sem_out):
        wid = lax.axis_index("s") * info.num_cores + lax.axis_index("c")
        base = wid * rows_per_worker

        def copy_in(i, b):
            return pltpu.async_copy(
                emb_hbm.at[pl.ds(base + i * _CHUNK_ROWS, _CHUNK_ROWS)],
                buf.at[b],
                sem_in,
            )

        def copy_out(i, b):
            return pltpu.async_copy(
                buf.at[b],
                out_hbm.at[pl.ds(base + i * _CHUNK_ROWS, _CHUNK_ROWS)],
                sem_out,
            )

        d_in = [None] * n_chunks
        d_out = [None] * n_chunks
        d_in[0] = copy_in(0, 0)
        if n_chunks > 1:
            d_in[1] = copy_in(1, 1)
        d_in[0].wait()
        d_out[0] = copy_out(0, 0)
        for i in range(1, n_chunks):
            d_in[i].wait()
            d_out[i] = copy_out(i, i % 2)
            if i + 1 < n_chunks:
                d_out[i - 1].wait()
                d_in[i + 1] = copy_in(i + 1, (i + 1) % 2)
        for i in range(max(0, n_chunks - 2), n_chunks):
            d_out[i].wait()

    return _sc_copy(embedding)[None]


# speedup vs baseline: 21.6144x; 1.0145x over previous
import functools

import jax
import jax.numpy as jnp
from jax import lax
from jax.experimental import pallas as pl
from jax.experimental.pallas import tpu as pltpu
from jax.experimental.pallas import tpu_sc as plsc

_CHUNK_ROWS = 32
_NBUF = 4


def kernel(x, embedding):
    seq_len = x.shape[1]
    d_model = embedding.shape[1]
    info = plsc.get_sparse_core_info()
    num_workers = info.num_cores * info.num_subcores
    rows_per_worker = seq_len // num_workers
    n_chunks = rows_per_worker // _CHUNK_ROWS

    mesh = plsc.VectorSubcoreMesh(core_axis_name="c", subcore_axis_name="s")

    @functools.partial(
        pl.kernel,
        mesh=mesh,
        out_type=jax.ShapeDtypeStruct((seq_len, d_model), embedding.dtype),
        scratch_types=[
            pltpu.VMEM((_NBUF, _CHUNK_ROWS, d_model), embedding.dtype),
            pltpu.SemaphoreType.DMA,
            pltpu.SemaphoreType.DMA,
        ],
    )
    def _sc_copy(emb_hbm, out_hbm, buf, sem_in, sem_out):
        wid = lax.axis_index("s") * info.num_cores + lax.axis_index("c")
        base = wid * rows_per_worker

        def copy_in(i):
            return pltpu.async_copy(
                emb_hbm.at[pl.ds(base + i * _CHUNK_ROWS, _CHUNK_ROWS)],
                buf.at[i % _NBUF],
                sem_in,
            )

        def copy_out(i):
            return pltpu.async_copy(
                buf.at[i % _NBUF],
                out_hbm.at[pl.ds(base + i * _CHUNK_ROWS, _CHUNK_ROWS)],
                sem_out,
            )

        d_in = [None] * n_chunks
        d_out = [None] * n_chunks
        for b in range(min(_NBUF - 1, n_chunks)):
            d_in[b] = copy_in(b)
        for i in range(n_chunks):
            d_in[i].wait()
            d_out[i] = copy_out(i)
            j = i + _NBUF - 1
            if j < n_chunks:
                if j - _NBUF >= 0:
                    d_out[j - _NBUF].wait()
                d_in[j] = copy_in(j)
        for i in range(max(0, n_chunks - _NBUF), n_chunks):
            d_out[i].wait()

    return _sc_copy(embedding)[None]
